# baseline (speedup 1.0000x reference)
import functools

import jax
import jax.numpy as jnp
from jax import lax
from jax.experimental import pallas as pl
from jax.experimental.pallas import tpu as pltpu
from jax.experimental.pallas import tpu_sc as plsc

N = 10000
E = 320000
D = 128
DA = 144
NC, NS = 2, 16
NW = NC * NS
EPT = E // NW
CH = 80
NCHUNK = EPT // CH
RPT = 624
TAIL = N - NS * RPT
ZR = 48


def _sc_aggregate(xa, packed):
  mesh = plsc.VectorSubcoreMesh(core_axis_name="c", subcore_axis_name="s")

  @functools.partial(
      pl.kernel,
      out_type=jax.ShapeDtypeStruct((NC * N, DA), jnp.float32),
      mesh=mesh,
      scratch_types=[
          pltpu.VMEM_SHARED((N, DA), jnp.float32),
          pltpu.VMEM((ZR, DA), jnp.float32),
          pltpu.VMEM((NCHUNK, CH), jnp.int32),
          pltpu.VMEM((CH,), jnp.int32),
          pltpu.VMEM((CH,), jnp.int32),
          pltpu.VMEM((CH,), jnp.int32),
          pltpu.VMEM((CH,), jnp.int32),
          pltpu.VMEM((CH, DA), jnp.float32),
          pltpu.VMEM((CH, DA), jnp.float32),
          pltpu.SemaphoreType.DMA,
          pltpu.SemaphoreType.DMA,
          pltpu.SemaphoreType.DMA,
          pltpu.SemaphoreType.DMA,
      ],
      compiler_params=pltpu.CompilerParams(use_tc_tiling_on_sc=False),
  )
  def body(xa_hbm, pk_hbm, out_hbm, acc, zbuf, pidx, sb0, sb1, db0, db1,
           rows0, rows1, gsem0, gsem1, ssem0, ssem1):
    c = lax.axis_index("c")
    s = lax.axis_index("s")
    wid = c * NS + s

    pltpu.async_copy(pk_hbm.at[pl.ds(wid * NCHUNK, NCHUNK)], pidx, gsem0)

    def zrow(i, carry):
      for j in range(DA // 16):
        zbuf[i, pl.ds(j * 16, 16)] = jnp.zeros((16,), jnp.float32)
      return carry
    lax.fori_loop(0, ZR, zrow, 0)
    pltpu.make_async_copy(pk_hbm.at[pl.ds(0, NCHUNK)], pidx, gsem0).wait()
    for t in range(RPT // ZR):
      pltpu.sync_copy(zbuf, acc.at[pl.ds(s * RPT + t * ZR, ZR)])
    @pl.when(s == 0)
    def _():
      pltpu.sync_copy(zbuf.at[pl.ds(0, TAIL)], acc.at[pl.ds(NS * RPT, TAIL)])
    plsc.subcore_barrier()

    def unpack(i, sb, db):
      def lane(j, carry):
        v = pidx[i, pl.ds(j * 16, 16)]
        sb[pl.ds(j * 16, 16)] = v & jnp.int32(0x3FFF)
        db[pl.ds(j * 16, 16)] = lax.shift_right_logical(v, jnp.int32(14))
        return carry
      lax.fori_loop(0, CH // 16, lane, 0)

    def gather(sb, buf, sem):
      pltpu.async_copy(xa_hbm.at[sb], buf, sem)

    def gwait(buf, sem):
      pltpu.make_async_copy(xa_hbm.at[sb0], buf, sem).wait()

    def sstart(db, buf, sem):
      pltpu.async_copy(buf, acc.at[db], sem, add=True)

    def swait(db, buf, sem):
      pltpu.make_async_copy(buf, acc.at[db], sem).wait()

    unpack(0, sb0, db0)
    gather(sb0, rows0, gsem0)
    unpack(1, sb1, db1)
    gather(sb1, rows1, gsem1)

    def step(k, carry):
      gwait(rows0, gsem0)
      sstart(db0, rows0, ssem0)
      gwait(rows1, gsem1)
      sstart(db1, rows1, ssem1)
      swait(db0, rows0, ssem0)
      unpack(2 * k + 2, sb0, db0)
      gather(sb0, rows0, gsem0)
      swait(db1, rows1, ssem1)
      @pl.when(2 * k + 3 < NCHUNK)
      def _():
        unpack(2 * k + 3, sb1, db1)
        gather(sb1, rows1, gsem1)
      return carry
    lax.fori_loop(0, NCHUNK // 2, step, 0)
    gwait(rows0, gsem0)
    sstart(db0, rows0, ssem0)
    swait(db0, rows0, ssem0)
    plsc.subcore_barrier()

    pltpu.sync_copy(acc.at[pl.ds(s * RPT, RPT)],
                    out_hbm.at[pl.ds(c * N + s * RPT, RPT)])
    @pl.when(s == 0)
    def _():
      pltpu.sync_copy(acc.at[pl.ds(NS * RPT, TAIL)],
                      out_hbm.at[pl.ds(c * N + NS * RPT, TAIL)])

  return body(xa, packed)


def _tc_finish(x, acc2, W_self, b_self, W_neigh, b_neigh):
  RB = 1000

  def body(x_ref, a0_ref, a1_ref, ws_ref, wn_ref, bs_ref, bn_ref, o_ref):
    ssum = a0_ref[...] + a1_ref[...]
    neigh = ssum[:, :D] / jnp.maximum(ssum[:, D:D + 1], 1.0)
    o_ref[...] = (
        jnp.dot(x_ref[...], ws_ref[...], preferred_element_type=jnp.float32)
        + jnp.dot(neigh, wn_ref[...], preferred_element_type=jnp.float32)
        + bs_ref[...] + bn_ref[...])

  grid = (N // RB,)
  return pl.pallas_call(
      body,
      grid=grid,
      in_specs=[
          pl.BlockSpec((RB, D), lambda i: (i, 0)),
          pl.BlockSpec((RB, DA), lambda i: (i, 0)),
          pl.BlockSpec((RB, DA), lambda i: (i + N // RB, 0)),
          pl.BlockSpec((D, D), lambda i: (0, 0)),
          pl.BlockSpec((D, D), lambda i: (0, 0)),
          pl.BlockSpec((1, D), lambda i: (0, 0)),
          pl.BlockSpec((1, D), lambda i: (0, 0)),
      ],
      out_specs=pl.BlockSpec((RB, D), lambda i: (i, 0)),
      out_shape=jax.ShapeDtypeStruct((N, D), jnp.float32),
  )(x, acc2, acc2, W_self, W_neigh, b_self, b_neigh)


def kernel(x, edge_index, W_self, b_self, W_neigh, b_neigh):
  packed = (edge_index[0] | (edge_index[1] << 14)).reshape(E // CH, CH)
  xa = jnp.concatenate(
      [x, jnp.ones((N, 1), jnp.float32), jnp.zeros((N, DA - D - 1), jnp.float32)],
      axis=1)
  acc2 = _sc_aggregate(xa, packed)
  return _tc_finish(x, acc2, W_self, b_self.reshape(1, D), W_neigh,
                    b_neigh.reshape(1, D))


# speedup vs baseline: 1.0095x; 1.0095x over previous
import functools

import jax
import jax.numpy as jnp
from jax import lax
from jax.experimental import pallas as pl
from jax.experimental.pallas import tpu as pltpu
from jax.experimental.pallas import tpu_sc as plsc

N = 10000
E = 320000
D = 128
DA = 144
NC, NS = 2, 16
NW = NC * NS
EPT = E // NW
CH = 80
NCHUNK = EPT // CH
RPT = 624
TAIL = N - NS * RPT
ZR = 48


def _sc_aggregate(xa, packed):
  mesh = plsc.VectorSubcoreMesh(core_axis_name="c", subcore_axis_name="s")

  @functools.partial(
      pl.kernel,
      out_type=jax.ShapeDtypeStruct((NC * N, DA), jnp.float32),
      mesh=mesh,
      scratch_types=[
          pltpu.VMEM_SHARED((N, DA), jnp.float32),
          pltpu.VMEM((ZR, DA), jnp.float32),
          pltpu.VMEM((NCHUNK, CH), jnp.int32),
          pltpu.VMEM((CH,), jnp.int32),
          pltpu.VMEM((CH,), jnp.int32),
          pltpu.VMEM((CH,), jnp.int32),
          pltpu.VMEM((CH,), jnp.int32),
          pltpu.VMEM((CH, DA), jnp.float32),
          pltpu.VMEM((CH, DA), jnp.float32),
          pltpu.SemaphoreType.DMA,
          pltpu.SemaphoreType.DMA,
          pltpu.SemaphoreType.DMA,
          pltpu.SemaphoreType.DMA,
      ],
      compiler_params=pltpu.CompilerParams(use_tc_tiling_on_sc=False),
  )
  def body(xa_hbm, pk_hbm, out_hbm, acc, zbuf, pidx, sb0, sb1, db0, db1,
           rows0, rows1, gsem0, gsem1, ssem0, ssem1):
    c = lax.axis_index("c")
    s = lax.axis_index("s")
    wid = c * NS + s

    pltpu.async_copy(pk_hbm.at[pl.ds(wid * NCHUNK, NCHUNK)], pidx, gsem0)

    def zrow(i, carry):
      for j in range(DA // 16):
        zbuf[i, pl.ds(j * 16, 16)] = jnp.zeros((16,), jnp.float32)
      return carry
    lax.fori_loop(0, ZR, zrow, 0)
    pltpu.make_async_copy(pk_hbm.at[pl.ds(0, NCHUNK)], pidx, gsem0).wait()
    for t in range(RPT // ZR):
      pltpu.sync_copy(zbuf, acc.at[pl.ds(s * RPT + t * ZR, ZR)])
    @pl.when(s == 0)
    def _():
      pltpu.sync_copy(zbuf.at[pl.ds(0, TAIL)], acc.at[pl.ds(NS * RPT, TAIL)])
    plsc.subcore_barrier()

    def unpack(i, sb, db):
      def lane(j, carry):
        v = pidx[i, pl.ds(j * 16, 16)]
        sb[pl.ds(j * 16, 16)] = v & jnp.int32(0x3FFF)
        db[pl.ds(j * 16, 16)] = lax.shift_right_logical(v, jnp.int32(14))
        return carry
      lax.fori_loop(0, CH // 16, lane, 0)

    def gather(sb, buf, sem):
      pltpu.async_copy(xa_hbm.at[sb], buf, sem)

    def gwait(buf, sem):
      pltpu.make_async_copy(xa_hbm.at[sb0], buf, sem).wait()

    def sstart(db, buf, sem):
      pltpu.async_copy(buf, acc.at[db], sem, add=True)

    def swait(db, buf, sem):
      pltpu.make_async_copy(buf, acc.at[db], sem).wait()

    unpack(0, sb0, db0)
    gather(sb0, rows0, gsem0)
    gwait(rows0, gsem0)
    sstart(db0, rows0, ssem0)
    unpack(1, sb1, db1)
    gather(sb1, rows1, gsem1)

    def step(k, carry):
      gwait(rows1, gsem1)
      sstart(db1, rows1, ssem1)
      swait(db0, rows0, ssem0)
      unpack(2 * k + 2, sb0, db0)
      gather(sb0, rows0, gsem0)
      gwait(rows0, gsem0)
      sstart(db0, rows0, ssem0)
      swait(db1, rows1, ssem1)
      @pl.when(2 * k + 3 < NCHUNK)
      def _():
        unpack(2 * k + 3, sb1, db1)
        gather(sb1, rows1, gsem1)
      return carry
    lax.fori_loop(0, NCHUNK // 2, step, 0)
    swait(db0, rows0, ssem0)
    plsc.subcore_barrier()

    pltpu.sync_copy(acc.at[pl.ds(s * RPT, RPT)],
                    out_hbm.at[pl.ds(c * N + s * RPT, RPT)])
    @pl.when(s == 0)
    def _():
      pltpu.sync_copy(acc.at[pl.ds(NS * RPT, TAIL)],
                      out_hbm.at[pl.ds(c * N + NS * RPT, TAIL)])

  return body(xa, packed)


def _tc_finish(x, acc2, W_self, b_self, W_neigh, b_neigh):
  RB = 1000

  def body(x_ref, a0_ref, a1_ref, ws_ref, wn_ref, bs_ref, bn_ref, o_ref):
    ssum = a0_ref[...] + a1_ref[...]
    neigh = ssum[:, :D] / jnp.maximum(ssum[:, D:D + 1], 1.0)
    o_ref[...] = (
        jnp.dot(x_ref[...], ws_ref[...], preferred_element_type=jnp.float32)
        + jnp.dot(neigh, wn_ref[...], preferred_element_type=jnp.float32)
        + bs_ref[...] + bn_ref[...])

  grid = (N // RB,)
  return pl.pallas_call(
      body,
      grid=grid,
      in_specs=[
          pl.BlockSpec((RB, D), lambda i: (i, 0)),
          pl.BlockSpec((RB, DA), lambda i: (i, 0)),
          pl.BlockSpec((RB, DA), lambda i: (i + N // RB, 0)),
          pl.BlockSpec((D, D), lambda i: (0, 0)),
          pl.BlockSpec((D, D), lambda i: (0, 0)),
          pl.BlockSpec((1, D), lambda i: (0, 0)),
          pl.BlockSpec((1, D), lambda i: (0, 0)),
      ],
      out_specs=pl.BlockSpec((RB, D), lambda i: (i, 0)),
      out_shape=jax.ShapeDtypeStruct((N, D), jnp.float32),
  )(x, acc2, acc2, W_self, W_neigh, b_self, b_neigh)


def kernel(x, edge_index, W_self, b_self, W_neigh, b_neigh):
  packed = (edge_index[0] | (edge_index[1] << 14)).reshape(E // CH, CH)
  xa = jnp.concatenate(
      [x, jnp.ones((N, 1), jnp.float32), jnp.zeros((N, DA - D - 1), jnp.float32)],
      axis=1)
  acc2 = _sc_aggregate(xa, packed)
  return _tc_finish(x, acc2, W_self, b_self.reshape(1, D), W_neigh,
                    b_neigh.reshape(1, D))


# speedup vs baseline: 1.2002x; 1.1890x over previous
import functools

import jax
import jax.numpy as jnp
from jax import lax
from jax.experimental import pallas as pl
from jax.experimental.pallas import tpu as pltpu
from jax.experimental.pallas import tpu_sc as plsc

N = 10000
E = 320000
D = 128
DA = 144
NC, NS = 2, 16
NW = NC * NS
EPT = E // NW
CH = 80
NCHUNK = EPT // CH
RPT = 624
TAIL = N - NS * RPT
ZR = 48


def _sc_aggregate(xa, packed):
  mesh = plsc.VectorSubcoreMesh(core_axis_name="c", subcore_axis_name="s")

  @functools.partial(
      pl.kernel,
      out_type=jax.ShapeDtypeStruct((NC * N, DA), jnp.float32),
      mesh=mesh,
      scratch_types=[
          pltpu.VMEM_SHARED((N, DA), jnp.float32),
          pltpu.VMEM((ZR, DA), jnp.float32),
          pltpu.VMEM((NCHUNK, CH), jnp.int32),
          pltpu.VMEM((CH,), jnp.int32),
          pltpu.VMEM((CH,), jnp.int32),
          pltpu.VMEM((CH,), jnp.int32),
          pltpu.VMEM((CH,), jnp.int32),
          pltpu.VMEM((CH, DA), jnp.float32),
          pltpu.VMEM((CH, DA), jnp.float32),
          pltpu.SemaphoreType.DMA,
          pltpu.SemaphoreType.DMA,
      ],
      compiler_params=pltpu.CompilerParams(use_tc_tiling_on_sc=False),
  )
  def body(xa_hbm, pk_hbm, out_hbm, acc, zbuf, pidx, sb0, sb1, db0, db1,
           rows0, rows1, gsem0, gsem1):
    c = lax.axis_index("c")
    s = lax.axis_index("s")
    wid = c * NS + s

    pltpu.async_copy(pk_hbm.at[pl.ds(wid * NCHUNK, NCHUNK)], pidx, gsem0)

    def zrow(i, carry):
      for j in range(DA // 16):
        zbuf[i, pl.ds(j * 16, 16)] = jnp.zeros((16,), jnp.float32)
      return carry
    lax.fori_loop(0, ZR, zrow, 0)
    pltpu.make_async_copy(pk_hbm.at[pl.ds(0, NCHUNK)], pidx, gsem0).wait()
    for t in range(RPT // ZR):
      pltpu.sync_copy(zbuf, acc.at[pl.ds(s * RPT + t * ZR, ZR)])
    @pl.when(s == 0)
    def _():
      pltpu.sync_copy(zbuf.at[pl.ds(0, TAIL)], acc.at[pl.ds(NS * RPT, TAIL)])
    plsc.subcore_barrier()

    def unpack(i, sb, db):
      def lane(j, carry):
        v = pidx[i, pl.ds(j * 16, 16)]
        sb[pl.ds(j * 16, 16)] = v & jnp.int32(0x3FFF)
        db[pl.ds(j * 16, 16)] = lax.shift_right_logical(v, jnp.int32(14))
        return carry
      lax.fori_loop(0, CH // 16, lane, 0)

    def gather(sb, buf, sem):
      pltpu.async_copy(xa_hbm.at[sb], buf, sem)

    def gwait(buf, sem):
      pltpu.make_async_copy(xa_hbm.at[sb0], buf, sem).wait()

    def scatter(db, buf):
      pltpu.sync_copy(buf, acc.at[db], add=True)

    unpack(0, sb0, db0)
    gather(sb0, rows0, gsem0)

    def step(k, carry):
      unpack(2 * k + 1, sb1, db1)
      gather(sb1, rows1, gsem1)
      gwait(rows0, gsem0)
      scatter(db0, rows0)
      unpack(2 * k + 2, sb0, db0)
      gather(sb0, rows0, gsem0)
      gwait(rows1, gsem1)
      scatter(db1, rows1)
      return carry
    lax.fori_loop(0, (NCHUNK - 1) // 2, step, 0)
    gwait(rows0, gsem0)
    scatter(db0, rows0)
    plsc.subcore_barrier()

    pltpu.sync_copy(acc.at[pl.ds(s * RPT, RPT)],
                    out_hbm.at[pl.ds(c * N + s * RPT, RPT)])
    @pl.when(s == 0)
    def _():
      pltpu.sync_copy(acc.at[pl.ds(NS * RPT, TAIL)],
                      out_hbm.at[pl.ds(c * N + NS * RPT, TAIL)])

  return body(xa, packed)


def _tc_finish(x, acc2, W_self, b_self, W_neigh, b_neigh):
  RB = 1000

  def body(x_ref, a0_ref, a1_ref, ws_ref, wn_ref, bs_ref, bn_ref, o_ref):
    ssum = a0_ref[...] + a1_ref[...]
    neigh = ssum[:, :D] / jnp.maximum(ssum[:, D:D + 1], 1.0)
    o_ref[...] = (
        jnp.dot(x_ref[...], ws_ref[...], preferred_element_type=jnp.float32)
        + jnp.dot(neigh, wn_ref[...], preferred_element_type=jnp.float32)
        + bs_ref[...] + bn_ref[...])

  grid = (N // RB,)
  return pl.pallas_call(
      body,
      grid=grid,
      in_specs=[
          pl.BlockSpec((RB, D), lambda i: (i, 0)),
          pl.BlockSpec((RB, DA), lambda i: (i, 0)),
          pl.BlockSpec((RB, DA), lambda i: (i + N // RB, 0)),
          pl.BlockSpec((D, D), lambda i: (0, 0)),
          pl.BlockSpec((D, D), lambda i: (0, 0)),
          pl.BlockSpec((1, D), lambda i: (0, 0)),
          pl.BlockSpec((1, D), lambda i: (0, 0)),
      ],
      out_specs=pl.BlockSpec((RB, D), lambda i: (i, 0)),
      out_shape=jax.ShapeDtypeStruct((N, D), jnp.float32),
  )(x, acc2, acc2, W_self, W_neigh, b_self, b_neigh)


def kernel(x, edge_index, W_self, b_self, W_neigh, b_neigh):
  packed = (edge_index[0] | (edge_index[1] << 14)).reshape(E // CH, CH)
  xa = jnp.concatenate(
      [x, jnp.ones((N, 1), jnp.float32), jnp.zeros((N, DA - D - 1), jnp.float32)],
      axis=1)
  acc2 = _sc_aggregate(xa, packed)
  return _tc_finish(x, acc2, W_self, b_self.reshape(1, D), W_neigh,
                    b_neigh.reshape(1, D))


# speedup vs baseline: 1.3460x; 1.1215x over previous
import functools

import jax
import jax.numpy as jnp
from jax import lax
from jax.experimental import pallas as pl
from jax.experimental.pallas import tpu as pltpu
from jax.experimental.pallas import tpu_sc as plsc

N = 10000
E = 320000
D = 128
DA = 144
NC, NS = 2, 16
NW = NC * NS
EPT = E // NW
CH = 80
NCHUNK = EPT // CH
RPT = 624
TAIL = N - NS * RPT
ZR = 48


def _sc_aggregate(xa, packed):
  mesh = plsc.VectorSubcoreMesh(core_axis_name="c", subcore_axis_name="s")

  @functools.partial(
      pl.kernel,
      out_type=(jax.ShapeDtypeStruct((NC * N, D), jnp.float32),
                jax.ShapeDtypeStruct((NC * N, DA - D), jnp.float32)),
      mesh=mesh,
      scratch_types=[
          pltpu.VMEM_SHARED((N, DA), jnp.float32),
          pltpu.VMEM((ZR, DA), jnp.float32),
          pltpu.VMEM((NCHUNK, CH), jnp.int32),
          pltpu.VMEM((CH,), jnp.int32),
          pltpu.VMEM((CH,), jnp.int32),
          pltpu.VMEM((CH,), jnp.int32),
          pltpu.VMEM((CH,), jnp.int32),
          pltpu.VMEM((CH, DA), jnp.float32),
          pltpu.VMEM((CH, DA), jnp.float32),
          pltpu.SemaphoreType.DMA,
          pltpu.SemaphoreType.DMA,
      ],
      compiler_params=pltpu.CompilerParams(use_tc_tiling_on_sc=False),
  )
  def body(xa_hbm, pk_hbm, feat_hbm, deg_hbm, acc, zbuf, pidx, sb0, sb1, db0,
           db1, rows0, rows1, gsem0, gsem1):
    c = lax.axis_index("c")
    s = lax.axis_index("s")
    wid = c * NS + s

    pltpu.async_copy(pk_hbm.at[pl.ds(wid * NCHUNK, NCHUNK)], pidx, gsem0)

    def zrow(i, carry):
      for j in range(DA // 16):
        zbuf[i, pl.ds(j * 16, 16)] = jnp.zeros((16,), jnp.float32)
      return carry
    lax.fori_loop(0, ZR, zrow, 0)
    pltpu.make_async_copy(pk_hbm.at[pl.ds(0, NCHUNK)], pidx, gsem0).wait()
    for t in range(RPT // ZR):
      pltpu.sync_copy(zbuf, acc.at[pl.ds(s * RPT + t * ZR, ZR)])
    @pl.when(s == 0)
    def _():
      pltpu.sync_copy(zbuf.at[pl.ds(0, TAIL)], acc.at[pl.ds(NS * RPT, TAIL)])
    plsc.subcore_barrier()

    def unpack(i, sb, db):
      def lane(j, carry):
        v = pidx[i, pl.ds(j * 16, 16)]
        sb[pl.ds(j * 16, 16)] = v & jnp.int32(0x3FFF)
        db[pl.ds(j * 16, 16)] = lax.shift_right_logical(v, jnp.int32(14))
        return carry
      lax.fori_loop(0, CH // 16, lane, 0)

    def gather(sb, buf, sem):
      pltpu.async_copy(xa_hbm.at[sb], buf, sem)

    def gwait(buf, sem):
      pltpu.make_async_copy(xa_hbm.at[sb0], buf, sem).wait()

    def scatter(db, buf):
      pltpu.sync_copy(buf, acc.at[db], add=True)

    unpack(0, sb0, db0)
    gather(sb0, rows0, gsem0)

    def step(k, carry):
      unpack(2 * k + 1, sb1, db1)
      gather(sb1, rows1, gsem1)
      gwait(rows0, gsem0)
      scatter(db0, rows0)
      unpack(2 * k + 2, sb0, db0)
      gather(sb0, rows0, gsem0)
      gwait(rows1, gsem1)
      scatter(db1, rows1)
      return carry
    lax.fori_loop(0, (NCHUNK - 1) // 2, step, 0)
    gwait(rows0, gsem0)
    scatter(db0, rows0)
    plsc.subcore_barrier()

    pltpu.sync_copy(acc.at[pl.ds(s * RPT, RPT), pl.ds(0, D)],
                    feat_hbm.at[pl.ds(c * N + s * RPT, RPT)])
    pltpu.sync_copy(acc.at[pl.ds(s * RPT, RPT), pl.ds(D, DA - D)],
                    deg_hbm.at[pl.ds(c * N + s * RPT, RPT)])
    @pl.when(s == 0)
    def _():
      pltpu.sync_copy(acc.at[pl.ds(NS * RPT, TAIL), pl.ds(0, D)],
                      feat_hbm.at[pl.ds(c * N + NS * RPT, TAIL)])
      pltpu.sync_copy(acc.at[pl.ds(NS * RPT, TAIL), pl.ds(D, DA - D)],
                      deg_hbm.at[pl.ds(c * N + NS * RPT, TAIL)])

  return body(xa, packed)


def _tc_finish(x, feat, deg, W_self, b_self, W_neigh, b_neigh):
  DGC = N * (DA - D) // 128

  def body(x_ref, f_ref, d_ref, ws_ref, wn_ref, bs_ref, bn_ref, o_ref):
    dsum = d_ref[:DGC, :] + d_ref[DGC:, :]
    d8 = jnp.repeat(dsum, 8, axis=0)
    lane = lax.broadcasted_iota(jnp.int32, (N, 128), 1)
    grp = lax.broadcasted_iota(jnp.int32, (N, 128), 0) % 8
    degcol = jnp.sum(jnp.where(lane == 16 * grp, d8, 0.0), axis=1,
                     keepdims=True)
    neigh = (f_ref[:N, :] + f_ref[N:, :]) / jnp.maximum(degcol, 1.0)
    o_ref[...] = (
        jnp.dot(x_ref[...], ws_ref[...], preferred_element_type=jnp.float32)
        + jnp.dot(neigh, wn_ref[...], preferred_element_type=jnp.float32)
        + bs_ref[...] + bn_ref[...])

  return pl.pallas_call(
      body,
      out_shape=jax.ShapeDtypeStruct((N, D), jnp.float32),
  )(x, feat, deg, W_self, W_neigh, b_self.reshape(1, D), b_neigh.reshape(1, D))


def kernel(x, edge_index, W_self, b_self, W_neigh, b_neigh):
  packed = (edge_index[0] | (edge_index[1] << 14)).reshape(E // CH, CH)
  xa = jnp.concatenate(
      [x, jnp.ones((N, 1), jnp.float32), jnp.zeros((N, DA - D - 1), jnp.float32)],
      axis=1)
  feat, deg = _sc_aggregate(xa, packed)
  deg = deg.reshape(NC * N * (DA - D) // 128, 128)
  return _tc_finish(x, feat, deg, W_self, b_self, W_neigh, b_neigh)


# speedup vs baseline: 1.3475x; 1.0011x over previous
import functools

import jax
import jax.numpy as jnp
from jax import lax
from jax.experimental import pallas as pl
from jax.experimental.pallas import tpu as pltpu
from jax.experimental.pallas import tpu_sc as plsc

N = 10000
E = 320000
D = 128
DA = 144
NC, NS = 2, 16
NW = NC * NS
EPT = E // NW
CH = 80
NCHUNK = EPT // CH
RPT = 624
TAIL = N - NS * RPT
ZR = 48


def _sc_aggregate(xa, packed):
  mesh = plsc.VectorSubcoreMesh(core_axis_name="c", subcore_axis_name="s")

  @functools.partial(
      pl.kernel,
      out_type=(jax.ShapeDtypeStruct((NC * N, D), jnp.float32),
                jax.ShapeDtypeStruct((NC * N, DA - D), jnp.float32)),
      mesh=mesh,
      scratch_types=[
          pltpu.VMEM_SHARED((N, DA), jnp.float32),
          pltpu.VMEM((ZR, DA), jnp.float32),
          pltpu.VMEM((EPT,), jnp.int32),
          pltpu.VMEM((CH,), jnp.int32),
          pltpu.VMEM((CH,), jnp.int32),
          pltpu.VMEM((CH,), jnp.int32),
          pltpu.VMEM((CH,), jnp.int32),
          pltpu.VMEM((CH, DA), jnp.float32),
          pltpu.VMEM((CH, DA), jnp.float32),
          pltpu.SemaphoreType.DMA,
          pltpu.SemaphoreType.DMA,
      ],
      compiler_params=pltpu.CompilerParams(use_tc_tiling_on_sc=False),
  )
  def body(xa_hbm, pk_hbm, feat_hbm, deg_hbm, acc, zbuf, pidx, sb0, sb1, db0,
           db1, rows0, rows1, gsem0, gsem1):
    c = lax.axis_index("c")
    s = lax.axis_index("s")
    wid = c * NS + s

    pltpu.async_copy(pk_hbm.at[pl.ds(wid * EPT, EPT)], pidx, gsem0)

    def zrow(i, carry):
      for j in range(DA // 16):
        zbuf[i, pl.ds(j * 16, 16)] = jnp.zeros((16,), jnp.float32)
      return carry
    lax.fori_loop(0, ZR, zrow, 0)
    pltpu.make_async_copy(pk_hbm.at[pl.ds(0, EPT)], pidx, gsem0).wait()
    for t in range(RPT // ZR):
      pltpu.sync_copy(zbuf, acc.at[pl.ds(s * RPT + t * ZR, ZR)])
    @pl.when(s == 0)
    def _():
      pltpu.sync_copy(zbuf.at[pl.ds(0, TAIL)], acc.at[pl.ds(NS * RPT, TAIL)])
    plsc.subcore_barrier()

    def unpack(i, sb, db):
      for j in range(CH // 16):
        v = pidx[pl.ds(i * CH + j * 16, 16)]
        sb[pl.ds(j * 16, 16)] = v & jnp.int32(0x3FFF)
        db[pl.ds(j * 16, 16)] = lax.shift_right_logical(v, jnp.int32(14))

    def gather(sb, buf, sem):
      pltpu.async_copy(xa_hbm.at[sb], buf, sem)

    def gwait(buf, sem):
      pltpu.make_async_copy(xa_hbm.at[sb0], buf, sem).wait()

    def scatter(db, buf):
      pltpu.sync_copy(buf, acc.at[db], add=True)

    unpack(0, sb0, db0)
    gather(sb0, rows0, gsem0)

    def step(k, carry):
      unpack(2 * k + 1, sb1, db1)
      gather(sb1, rows1, gsem1)
      gwait(rows0, gsem0)
      scatter(db0, rows0)
      unpack(2 * k + 2, sb0, db0)
      gather(sb0, rows0, gsem0)
      gwait(rows1, gsem1)
      scatter(db1, rows1)
      return carry
    lax.fori_loop(0, (NCHUNK - 1) // 2, step, 0)
    gwait(rows0, gsem0)
    scatter(db0, rows0)
    plsc.subcore_barrier()

    pltpu.sync_copy(acc.at[pl.ds(s * RPT, RPT), pl.ds(0, D)],
                    feat_hbm.at[pl.ds(c * N + s * RPT, RPT)])
    pltpu.sync_copy(acc.at[pl.ds(s * RPT, RPT), pl.ds(D, DA - D)],
                    deg_hbm.at[pl.ds(c * N + s * RPT, RPT)])
    @pl.when(s == 0)
    def _():
      pltpu.sync_copy(acc.at[pl.ds(NS * RPT, TAIL), pl.ds(0, D)],
                      feat_hbm.at[pl.ds(c * N + NS * RPT, TAIL)])
      pltpu.sync_copy(acc.at[pl.ds(NS * RPT, TAIL), pl.ds(D, DA - D)],
                      deg_hbm.at[pl.ds(c * N + NS * RPT, TAIL)])

  return body(xa, packed)


def _tc_finish(x, feat, deg, W_self, b_self, W_neigh, b_neigh):
  DGC = N * (DA - D) // 128

  def body(x_ref, f_ref, d_ref, ws_ref, wn_ref, bs_ref, bn_ref, o_ref):
    dsum = d_ref[:DGC, :] + d_ref[DGC:, :]
    d8 = jnp.repeat(dsum, 8, axis=0)
    lane = lax.broadcasted_iota(jnp.int32, (N, 128), 1)
    grp = lax.broadcasted_iota(jnp.int32, (N, 128), 0) % 8
    degcol = jnp.sum(jnp.where(lane == 16 * grp, d8, 0.0), axis=1,
                     keepdims=True)
    neigh = (f_ref[:N, :] + f_ref[N:, :]) / jnp.maximum(degcol, 1.0)
    o_ref[...] = (
        jnp.dot(x_ref[...], ws_ref[...], preferred_element_type=jnp.float32)
        + jnp.dot(neigh, wn_ref[...], preferred_element_type=jnp.float32)
        + bs_ref[...] + bn_ref[...])

  return pl.pallas_call(
      body,
      out_shape=jax.ShapeDtypeStruct((N, D), jnp.float32),
  )(x, feat, deg, W_self, W_neigh, b_self.reshape(1, D), b_neigh.reshape(1, D))


def kernel(x, edge_index, W_self, b_self, W_neigh, b_neigh):
  packed = edge_index[0] | (edge_index[1] << 14)
  xa = jnp.concatenate(
      [x, jnp.ones((N, 1), jnp.float32), jnp.zeros((N, DA - D - 1), jnp.float32)],
      axis=1)
  feat, deg = _sc_aggregate(xa, packed)
  deg = deg.reshape(NC * N * (DA - D) // 128, 128)
  return _tc_finish(x, feat, deg, W_self, b_self, W_neigh, b_neigh)
